# Initial kernel scaffold; baseline (speedup 1.0000x reference)
import jax
import jax.numpy as jnp
from jax.experimental import pallas as pl


def kernel(x, edge_index, batch_data, W1, b1, W2, b2, W3, b3, Wl, bl):
    raise NotImplementedError("write your pallas kernel here")


# speedup vs baseline: 4.5710x; 4.5710x over previous
import functools

import jax
import jax.numpy as jnp
from jax import lax
from jax.experimental import pallas as pl
from jax.experimental.pallas import tpu as pltpu
from jax.experimental.pallas import tpu_sc as plsc

N = 50000
E = 800000
G = 256
H = 128

NP = 50176
EP = 802816
CPT = 392
STRIPE = NP // 16
NSLAB = 16
SLABW = H // NSLAB
NBUF = 4
NPAIR = CPT // NBUF
BLK = 1024
NB = NP // BLK

_SC_MESH = plsc.VectorSubcoreMesh(core_axis_name="c", subcore_axis_name="s")


@functools.partial(
    pl.kernel,
    mesh=_SC_MESH,
    compiler_params=pltpu.CompilerParams(use_tc_tiling_on_sc=False),
    out_type=jax.ShapeDtypeStruct((2, NP, 8), jnp.float32),
    scratch_types=[
        pltpu.VMEM((CPT, 128), jnp.int32),
        pltpu.VMEM((128, 8), jnp.float32),
        pltpu.VMEM_SHARED((NP, 8), jnp.float32),
    ],
)
def _k_deg(dst_hbm, zro_hbm, one_hbm, out_hbm, didx, ones_v, acc):
    cid = lax.axis_index("c")
    sid = lax.axis_index("s")
    stripe0 = pl.multiple_of(sid * STRIPE, STRIPE)
    pltpu.sync_copy(one_hbm, ones_v)
    pltpu.sync_copy(dst_hbm.at[sid], didx)
    pltpu.sync_copy(zro_hbm, acc.at[pl.ds(stripe0, STRIPE)])
    plsc.subcore_barrier()

    base = cid * (CPT // 2)

    def chunk(j, _):
        pltpu.sync_copy(ones_v, acc.at[didx.at[j]], add=True)
        return 0

    lax.fori_loop(base, base + CPT // 2, chunk, 0)
    plsc.subcore_barrier()
    pltpu.sync_copy(acc.at[pl.ds(stripe0, STRIPE)],
                    out_hbm.at[cid, pl.ds(stripe0, STRIPE)])


@functools.partial(
    pl.kernel,
    mesh=_SC_MESH,
    compiler_params=pltpu.CompilerParams(use_tc_tiling_on_sc=False),
    out_type=jax.ShapeDtypeStruct((NSLAB, NP, SLABW), jnp.float32),
    scratch_types=[
        pltpu.VMEM((CPT, 128), jnp.int32),
        pltpu.VMEM((CPT, 128), jnp.int32),
        pltpu.VMEM((128, SLABW), jnp.float32),
        pltpu.VMEM((128, SLABW), jnp.float32),
        pltpu.VMEM((128, SLABW), jnp.float32),
        pltpu.VMEM((128, SLABW), jnp.float32),
        pltpu.VMEM_SHARED((NP, SLABW), jnp.float32),
        pltpu.SemaphoreType.DMA,
        pltpu.SemaphoreType.DMA,
        pltpu.SemaphoreType.DMA,
        pltpu.SemaphoreType.DMA,
        pltpu.SemaphoreType.DMA,
        pltpu.SemaphoreType.DMA,
        pltpu.SemaphoreType.DMA,
        pltpu.SemaphoreType.DMA,
    ],
)
def _k_edge(y_hbm, src_hbm, dst_hbm, zro_hbm, z_hbm, sidx, didx,
            b0, b1, b2, b3, acc, g0, g1, g2, g3, s0, s1, s2, s3):
    bufs = (b0, b1, b2, b3)
    gsems = (g0, g1, g2, g3)
    ssems = (s0, s1, s2, s3)
    cid = lax.axis_index("c")
    sid = lax.axis_index("s")
    stripe0 = pl.multiple_of(sid * STRIPE, STRIPE)
    pltpu.sync_copy(src_hbm.at[sid], sidx)
    pltpu.sync_copy(dst_hbm.at[sid], didx)

    def slab(il, _):
        sl = cid * (NSLAB // 2) + il
        pltpu.sync_copy(zro_hbm, acc.at[pl.ds(stripe0, STRIPE)])
        plsc.subcore_barrier()

        for b in range(NBUF):
            pltpu.async_copy(y_hbm.at[sl].at[sidx.at[b]], bufs[b], gsems[b])

        def group(t, _):
            for b in range(NBUF):
                j = NBUF * t + b
                pltpu.make_async_copy(
                    y_hbm.at[sl].at[sidx.at[j]], bufs[b], gsems[b]).wait()
                pltpu.async_copy(bufs[b], acc.at[didx.at[j]], ssems[b],
                                 add=True)
            for b in range(NBUF):
                j = NBUF * t + b
                pltpu.make_async_copy(
                    bufs[b], acc.at[didx.at[j]], ssems[b]).wait()

                @pl.when(t < NPAIR - 1)
                def _():
                    pltpu.async_copy(
                        y_hbm.at[sl].at[sidx.at[j + NBUF]], bufs[b], gsems[b])

            return 0

        lax.fori_loop(0, NPAIR, group, 0)
        plsc.subcore_barrier()
        pltpu.sync_copy(acc.at[pl.ds(stripe0, STRIPE)],
                        z_hbm.at[sl, pl.ds(stripe0, STRIPE)])
        plsc.subcore_barrier()
        return 0

    lax.fori_loop(0, NSLAB // 2, slab, 0)


@functools.partial(
    pl.kernel,
    mesh=_SC_MESH,
    compiler_params=pltpu.CompilerParams(use_tc_tiling_on_sc=False),
    out_type=jax.ShapeDtypeStruct((G, H), jnp.float32),
    scratch_types=[
        pltpu.VMEM((2, G + 16), jnp.float32),
        pltpu.VMEM((128, H), jnp.float32),
        pltpu.VMEM((8, H), jnp.float32),
    ],
)
def _k_pool(h_hbm, se_hbm, out_hbm, sev, buf, orow):
    cid = lax.axis_index("c")
    sid = lax.axis_index("s")
    wid = sid * 2 + cid
    pltpu.sync_copy(se_hbm, sev.at[:, pl.ds(0, G)])
    g0 = pl.multiple_of(wid * 8, 8)
    neg = jnp.float32(-3.4e38)
    svec = sev[0, pl.ds(g0, 16)]
    evec = sev[1, pl.ds(g0, 16)]

    for gi in range(8):
        s_i = svec[gi].astype(jnp.int32)
        e_i = evec[gi].astype(jnp.int32)
        a_i = (s_i // 8) * 8
        nch = (e_i - a_i + 127) // 128
        nch = jnp.where(e_i > s_i, nch, 0)
        acc0 = tuple(jnp.full((16,), -jnp.inf, jnp.float32) for _ in range(8))

        def chunk(c, accs):
            r0 = pl.multiple_of(a_i + c * 128, 8)
            pltpu.sync_copy(h_hbm.at[pl.ds(r0, 128)], buf)

            def row(r, accs2):
                gr = r0 + r
                ok = jnp.logical_and(gr >= s_i, gr < e_i)
                pen = jnp.where(ok, jnp.float32(0.0), neg)
                out = []
                for k in range(8):
                    v = buf[r, pl.ds(k * 16, 16)] + pen
                    out.append(jnp.maximum(accs2[k], v))
                return tuple(out)

            return lax.fori_loop(0, 128, row, accs)

        accs = lax.fori_loop(0, nch, chunk, acc0)
        for k in range(8):
            orow[gi, pl.ds(k * 16, 16)] = accs[k]
    pltpu.sync_copy(orow, out_hbm.at[pl.ds(g0, 8)])


def _prep_body(x_ref, ind_ref, bat_ref, w_ref, y_ref, dinv_ref, se_ref):
    i = pl.program_id(0)
    deg8 = ind_ref[0] + ind_ref[1] + 1.0
    rows = i * BLK + lax.broadcasted_iota(jnp.int32, (BLK, 8), 0)
    dinv = jnp.where(rows < N, lax.rsqrt(deg8), 0.0)
    dinv_ref[...] = dinv
    xw = jnp.dot(x_ref[...], w_ref[...], preferred_element_type=jnp.float32)
    for s in range(NSLAB):
        y_ref[s] = xw[:, s * SLABW:(s + 1) * SLABW] * dinv

    b = bat_ref[...]
    gio = lax.broadcasted_iota(jnp.int32, (8, 128, G), 2)
    b3 = b[:, :, None]
    lt = jnp.sum(jnp.sum((b3 < gio).astype(jnp.float32), axis=0), axis=0)
    le = jnp.sum(jnp.sum((b3 <= gio).astype(jnp.float32), axis=0), axis=0)

    @pl.when(i == 0)
    def _():
        se_ref[...] = jnp.zeros((2, G), jnp.float32)

    se_ref[...] = se_ref[...] + jnp.stack([lt, le])


def _relu_layer(z_ref, y_ref, dinv, b_ref):
    parts = []
    for s in range(NSLAB):
        t = (z_ref[s] + y_ref[s]) * dinv + b_ref[:, s * SLABW:(s + 1) * SLABW]
        parts.append(jnp.maximum(t, 0.0))
    return jnp.concatenate(parts, axis=1)


def _comb_body(z_ref, y_ref, dinv_ref, b_ref, w_ref, o_ref, h_ref):
    dinv = dinv_ref[...]
    h = _relu_layer(z_ref, y_ref, dinv, b_ref)
    h_ref[...] = h
    yn = jnp.dot(h, w_ref[...], preferred_element_type=jnp.float32)
    for s in range(NSLAB):
        o_ref[s] = yn[:, s * SLABW:(s + 1) * SLABW] * dinv


def _head_body(p_ref, w_ref, bl_ref, o_ref):
    v = jnp.dot(p_ref[...], w_ref[...], preferred_element_type=jnp.float32)
    v = v + bl_ref[0, 0]
    o_ref[...] = jnp.where(v > 0.0, v, jnp.exp(v) - 1.0)


_SLAB_SPEC = pl.BlockSpec((NSLAB, BLK, SLABW), lambda i: (0, i, 0))
_DINV_SPEC = pl.BlockSpec((BLK, SLABW), lambda i: (i, 0))
_W_SPEC = pl.BlockSpec((H, H), lambda i: (0, 0))
_B_SPEC = pl.BlockSpec((1, H), lambda i: (0, 0))

_prep_call = pl.pallas_call(
    _prep_body,
    grid=(NB,),
    in_specs=[
        pl.BlockSpec((BLK, 7), lambda i: (i, 0)),
        pl.BlockSpec((2, BLK, 8), lambda i: (0, i, 0)),
        pl.BlockSpec((8, 128), lambda i: (i, 0)),
        pl.BlockSpec((7, H), lambda i: (0, 0)),
    ],
    out_specs=[
        _SLAB_SPEC,
        _DINV_SPEC,
        pl.BlockSpec((2, G), lambda i: (0, 0)),
    ],
    out_shape=[
        jax.ShapeDtypeStruct((NSLAB, NP, SLABW), jnp.float32),
        jax.ShapeDtypeStruct((NP, SLABW), jnp.float32),
        jax.ShapeDtypeStruct((2, G), jnp.float32),
    ],
)

_comb_call = pl.pallas_call(
    _comb_body,
    grid=(NB,),
    in_specs=[_SLAB_SPEC, _SLAB_SPEC, _DINV_SPEC, _B_SPEC, _W_SPEC],
    out_specs=[_SLAB_SPEC, pl.BlockSpec((BLK, H), lambda i: (i, 0))],
    out_shape=[
        jax.ShapeDtypeStruct((NSLAB, NP, SLABW), jnp.float32),
        jax.ShapeDtypeStruct((NP, H), jnp.float32),
    ],
)

_head_call = pl.pallas_call(
    _head_body,
    grid=(1,),
    in_specs=[
        pl.BlockSpec((G, H), lambda i: (0, 0)),
        pl.BlockSpec((H, 1), lambda i: (0, 0)),
        pl.BlockSpec((1, 1), lambda i: (0, 0), memory_space=pltpu.SMEM),
    ],
    out_specs=pl.BlockSpec((G, 1), lambda i: (0, 0)),
    out_shape=jax.ShapeDtypeStruct((G, 1), jnp.float32),
)


def kernel(x, edge_index, batch_data, W1, b1, W2, b2, W3, b3, Wl, bl):
    src = edge_index[0]
    dst = edge_index[1]
    xp = jnp.zeros((NP, 7), jnp.float32).at[:N].set(x)
    pad_idx = jnp.full((EP - E,), N, jnp.int32)
    srcp = jnp.concatenate([src, pad_idx]).reshape(16, CPT, 128)
    dstp = jnp.concatenate([dst, pad_idx]).reshape(16, CPT, 128)
    batp = jnp.concatenate(
        [batch_data, jnp.full((NP - N,), G, jnp.int32)]).reshape(392, 128)
    zeros8 = jnp.zeros((STRIPE, 8), jnp.float32)
    ones8 = jnp.ones((128, 8), jnp.float32)

    indeg2 = _k_deg(dstp, zeros8, ones8)
    y1, dinv, se = _prep_call(xp, indeg2, batp, W1)

    bs = jnp.stack([b1.reshape(1, H), b2.reshape(1, H), b3.reshape(1, H)])
    Ws = jnp.stack([W2, W3, W3])

    def _layer(y, bw):
        b, W = bw
        z = _k_edge(y, srcp, dstp, zeros8)
        yn, h = _comb_call(z, y, dinv, b, W)
        return yn, h

    _, hs = lax.scan(_layer, y1, (bs, Ws))
    pooled = _k_pool(hs[2], se)
    out = _head_call(pooled, Wl, bl.reshape(1, 1))
    return out.reshape(G)
